# Initial kernel scaffold; baseline (speedup 1.0000x reference)
import jax
import jax.numpy as jnp
from jax.experimental import pallas as pl


def kernel(species_ids, move_ids, item_ids, ability_ids, species_table, move_table, item_table, ability_table):
    raise NotImplementedError("write your pallas kernel here")


# speedup vs baseline: 2.5850x; 2.5850x over previous
import functools

import jax
import jax.numpy as jnp
from jax import lax
from jax.experimental import pallas as pl
from jax.experimental.pallas import tpu as pltpu
from jax.experimental.pallas import tpu_sc as plsc

B = 4096
D = 64
N_SP, N_MV, N_IT, N_AB = 6, 24, 6, 6
OFF_SP, OFF_MV, OFF_IT, OFF_AB = 0, 6, 30, 36
N_ALL = N_SP + N_MV + N_IT + N_AB

NC, NS = 2, 16
NW = NC * NS
RW = B // NW
RC = 16
NCHUNK = RW // RC
BUF_BYTES = RC * N_ALL * D * 4


IROW = 48
IO_SP, IO_MV, IO_IT, IO_AB = 0, 8, 32, 40


def _body(ids, sp_tab, mv_tab, it_tab, ab_tab, out,
          idx, buf0, buf1, gsem, osem):
  wid = lax.axis_index("s") * NC + lax.axis_index("c")
  base = wid * RW

  pltpu.sync_copy(ids.at[pl.ds(base * IROW, RW * IROW)], idx)

  bufs = (buf0, buf1)

  def drain(sem, buf):
    pltpu.make_async_copy(out.at[pl.ds(0, RC)], buf, sem).wait()

  for c in range(NCHUNK):
    buf = bufs[c % 2]
    if c >= 2:
      drain(osem, buf)

    def fire(r, _):
      o = (c * RC + r) * IROW
      pltpu.async_copy(
          sp_tab.at[idx.at[pl.ds(o + IO_SP, N_SP)]],
          buf.at[r, pl.ds(OFF_SP, N_SP)], gsem)
      pltpu.async_copy(
          mv_tab.at[idx.at[pl.ds(o + IO_MV, N_MV)]],
          buf.at[r, pl.ds(OFF_MV, N_MV)], gsem)
      pltpu.async_copy(
          it_tab.at[idx.at[pl.ds(o + IO_IT, N_IT)]],
          buf.at[r, pl.ds(OFF_IT, N_IT)], gsem)
      pltpu.async_copy(
          ab_tab.at[idx.at[pl.ds(o + IO_AB, N_AB)]],
          buf.at[r, pl.ds(OFF_AB, N_AB)], gsem)
      return _

    lax.fori_loop(0, RC, fire, 0)
    drain(gsem, buf)
    pltpu.async_copy(buf, out.at[pl.ds(base + c * RC, RC)], osem)

  drain(osem, bufs[0])
  drain(osem, bufs[1])


@jax.jit
def _encode(ids, sp_tab, mv_tab, it_tab, ab_tab):
  mesh = plsc.VectorSubcoreMesh(core_axis_name="c", subcore_axis_name="s")
  f = pl.kernel(
      _body,
      out_type=jax.ShapeDtypeStruct((B, N_ALL, D), jnp.float32),
      mesh=mesh,
      compiler_params=pltpu.CompilerParams(use_tc_tiling_on_sc=False),
      scratch_types=[
          pltpu.VMEM((RW * IROW,), jnp.int32),
          pltpu.VMEM((RC, N_ALL, D), jnp.float32),
          pltpu.VMEM((RC, N_ALL, D), jnp.float32),
          pltpu.SemaphoreType.DMA,
          pltpu.SemaphoreType.DMA,
      ],
  )
  out = f(ids, sp_tab, mv_tab, it_tab, ab_tab)
  return out.reshape(B, N_ALL * D)


def kernel(species_ids, move_ids, item_ids, ability_ids,
           species_table, move_table, item_table, ability_table):
  ids = jnp.zeros((B, IROW), jnp.int32)
  ids = ids.at[:, IO_SP:IO_SP + N_SP].set(species_ids.astype(jnp.int32))
  ids = ids.at[:, IO_MV:IO_MV + N_MV].set(move_ids.astype(jnp.int32))
  ids = ids.at[:, IO_IT:IO_IT + N_IT].set(item_ids.astype(jnp.int32))
  ids = ids.at[:, IO_AB:IO_AB + N_AB].set(ability_ids.astype(jnp.int32))
  return _encode(ids.reshape(-1), species_table, move_table,
                 item_table, ability_table)
